# baseline (speedup 1.0000x reference)
import functools

import jax
import jax.numpy as jnp
from jax import lax
from jax.experimental import pallas as pl
from jax.experimental.pallas import tpu as pltpu
from jax.experimental.pallas import tpu_sc as plsc

NUM_WORKERS = 32
CHUNK = 128


def _make_gather(n_ids: int, vocab: int, dim: int):
  n_per_w = n_ids // NUM_WORKERS
  n_chunks = n_per_w // CHUNK
  mesh = plsc.VectorSubcoreMesh(core_axis_name="c", subcore_axis_name="s")

  @functools.partial(
      pl.kernel,
      mesh=mesh,
      out_type=jax.ShapeDtypeStruct((n_ids, dim), jnp.float32),
      scratch_types=[
          pltpu.VMEM((n_chunks, CHUNK), jnp.int32),
          pltpu.VMEM((CHUNK, dim), jnp.float32),
          pltpu.SemaphoreType.DMA,
      ],
      compiler_params=pltpu.CompilerParams(use_tc_tiling_on_sc=False),
  )
  def gather_kernel(ids_hbm, table_hbm, out_hbm, idx_v, rows_v, sem):
    c = lax.axis_index("c")
    s = lax.axis_index("s")
    wid = s * 2 + c
    pltpu.sync_copy(ids_hbm.at[wid], idx_v)

    def body(j, carry):
      pltpu.async_copy(table_hbm.at[idx_v.at[j]], rows_v, sem).wait()
      pltpu.sync_copy(
          rows_v, out_hbm.at[pl.ds(wid * n_per_w + j * CHUNK, CHUNK)])
      return carry

    lax.fori_loop(0, n_chunks, body, 0)

  return gather_kernel


def kernel(input_ids, attention_mask, emb_weight):
  batch, seq = input_ids.shape
  vocab, dim = emb_weight.shape
  n_ids = batch * seq
  ids = input_ids.reshape(NUM_WORKERS, n_ids // (NUM_WORKERS * CHUNK), CHUNK)
  ids = ids.astype(jnp.int32)
  out = _make_gather(n_ids, vocab, dim)(ids, emb_weight)
  return out.reshape(batch, seq, dim), attention_mask


# speedup vs baseline: 1.1160x; 1.1160x over previous
import functools

import jax
import jax.numpy as jnp
from jax import lax
from jax.experimental import pallas as pl
from jax.experimental.pallas import tpu as pltpu
from jax.experimental.pallas import tpu_sc as plsc

NUM_WORKERS = 32
CHUNK = 128
NBUF = 8


def _make_gather(n_ids: int, vocab: int, dim: int):
  n_per_w = n_ids // NUM_WORKERS
  n_chunks = n_per_w // CHUNK
  mesh = plsc.VectorSubcoreMesh(core_axis_name="c", subcore_axis_name="s")

  @functools.partial(
      pl.kernel,
      mesh=mesh,
      out_type=jax.ShapeDtypeStruct((n_ids, dim), jnp.float32),
      scratch_types=[
          pltpu.VMEM((n_chunks, CHUNK), jnp.int32),
          pltpu.VMEM((NBUF, CHUNK, dim), jnp.float32),
          pltpu.SemaphoreType.DMA((NBUF,)),
      ],
      compiler_params=pltpu.CompilerParams(use_tc_tiling_on_sc=False),
  )
  def gather_kernel(ids_hbm, table_hbm, out_hbm, idx_v, rows_v, gsem):
    c = lax.axis_index("c")
    s = lax.axis_index("s")
    wid = s * 2 + c
    base = wid * n_per_w
    pltpu.sync_copy(ids_hbm.at[wid], idx_v)

    for b in range(NBUF):
      pltpu.async_copy(table_hbm.at[idx_v.at[b]], rows_v.at[b], gsem.at[b])

    def body(j, carry):
      slot = lax.rem(j, NBUF)
      pltpu.make_async_copy(
          table_hbm.at[idx_v.at[j]], rows_v.at[slot], gsem.at[slot]).wait()
      pltpu.sync_copy(rows_v.at[slot], out_hbm.at[pl.ds(base + j * CHUNK, CHUNK)])
      nxt = j + NBUF
      pltpu.async_copy(table_hbm.at[idx_v.at[nxt]], rows_v.at[slot],
                       gsem.at[slot])
      return carry

    lax.fori_loop(0, n_chunks - NBUF, body, 0)

    for b in range(NBUF):
      j = n_chunks - NBUF + b
      slot = j % NBUF
      pltpu.make_async_copy(
          table_hbm.at[idx_v.at[j]], rows_v.at[slot], gsem.at[slot]).wait()
      pltpu.sync_copy(rows_v.at[slot],
                      out_hbm.at[pl.ds(base + j * CHUNK, CHUNK)])

  return gather_kernel


def kernel(input_ids, attention_mask, emb_weight):
  batch, seq = input_ids.shape
  vocab, dim = emb_weight.shape
  n_ids = batch * seq
  ids = input_ids.reshape(NUM_WORKERS, n_ids // (NUM_WORKERS * CHUNK), CHUNK)
  ids = ids.astype(jnp.int32)
  out = _make_gather(n_ids, vocab, dim)(ids, emb_weight)
  return out.reshape(batch, seq, dim), attention_mask


# speedup vs baseline: 1.3602x; 1.2187x over previous
import functools

import jax
import jax.numpy as jnp
from jax import lax
from jax.experimental import pallas as pl
from jax.experimental.pallas import tpu as pltpu
from jax.experimental.pallas import tpu_sc as plsc

NUM_WORKERS = 32
CHUNK = 128
NBUF = 4


def _make_gather(n_ids: int, vocab: int, padded_dim: int):
  n_per_w = n_ids // NUM_WORKERS
  n_chunks = n_per_w // CHUNK
  mesh = plsc.VectorSubcoreMesh(core_axis_name="c", subcore_axis_name="s")

  @functools.partial(
      pl.kernel,
      mesh=mesh,
      out_type=jax.ShapeDtypeStruct((n_ids, padded_dim), jnp.float32),
      scratch_types=[
          pltpu.VMEM((n_chunks, CHUNK), jnp.int32),
          pltpu.VMEM((NBUF, CHUNK, padded_dim), jnp.float32),
          pltpu.SemaphoreType.DMA((NBUF,)),
      ],
  )
  def gather_kernel(ids_hbm, table_hbm, out_hbm, idx_v, rows_v, gsem):
    c = lax.axis_index("c")
    s = lax.axis_index("s")
    wid = s * 2 + c
    base = wid * n_per_w
    pltpu.sync_copy(ids_hbm.at[wid], idx_v)

    for b in range(NBUF):
      pltpu.async_copy(table_hbm.at[idx_v.at[b]], rows_v.at[b], gsem.at[b])

    def body(j, carry):
      slot = lax.rem(j, NBUF)
      pltpu.make_async_copy(
          table_hbm.at[idx_v.at[j]], rows_v.at[slot], gsem.at[slot]).wait()
      pltpu.sync_copy(rows_v.at[slot], out_hbm.at[pl.ds(base + j * CHUNK, CHUNK)])
      nxt = j + NBUF
      pltpu.async_copy(table_hbm.at[idx_v.at[nxt]], rows_v.at[slot],
                       gsem.at[slot])
      return carry

    lax.fori_loop(0, n_chunks - NBUF, body, 0)

    for b in range(NBUF):
      j = n_chunks - NBUF + b
      slot = j % NBUF
      pltpu.make_async_copy(
          table_hbm.at[idx_v.at[j]], rows_v.at[slot], gsem.at[slot]).wait()
      pltpu.sync_copy(rows_v.at[slot],
                      out_hbm.at[pl.ds(base + j * CHUNK, CHUNK)])

  return gather_kernel


def kernel(input_ids, attention_mask, emb_weight):
  batch, seq = input_ids.shape
  vocab, dim = emb_weight.shape
  n_ids = batch * seq
  table128 = jnp.pad(emb_weight, ((0, 0), (0, 128 - dim)))
  ids = input_ids.reshape(NUM_WORKERS, n_ids // (NUM_WORKERS * CHUNK), CHUNK)
  ids = ids.astype(jnp.int32)
  out128 = _make_gather(n_ids, vocab, 128)(ids, table128)
  return out128[:, :dim].reshape(batch, seq, dim), attention_mask


# speedup vs baseline: 1.4803x; 1.0884x over previous
import functools

import jax
import jax.numpy as jnp
from jax import lax
from jax.experimental import pallas as pl
from jax.experimental.pallas import tpu as pltpu
from jax.experimental.pallas import tpu_sc as plsc

NUM_WORKERS = 32
CHUNK = 128
NBUF = 4


def _make_gather(n_ids: int, vocab: int, padded_dim: int):
  n_per_w = n_ids // NUM_WORKERS
  n_chunks = n_per_w // CHUNK
  mesh = plsc.VectorSubcoreMesh(core_axis_name="c", subcore_axis_name="s")

  dim = 64

  @functools.partial(
      pl.kernel,
      mesh=mesh,
      out_type=jax.ShapeDtypeStruct((n_ids, padded_dim), jnp.float32),
      scratch_types=[
          pltpu.VMEM((n_chunks, CHUNK), jnp.int32),
          pltpu.VMEM((NBUF, CHUNK, dim), jnp.float32),
          pltpu.SemaphoreType.DMA((NBUF,)),
      ],
      compiler_params=pltpu.CompilerParams(use_tc_tiling_on_sc=False),
  )
  def gather_kernel(ids_hbm, table_hbm, out_hbm, idx_v, rows_v, gsem):
    c = lax.axis_index("c")
    s = lax.axis_index("s")
    wid = s * 2 + c
    base = wid * n_per_w
    pltpu.sync_copy(ids_hbm.at[wid], idx_v)

    for b in range(NBUF):
      pltpu.async_copy(table_hbm.at[idx_v.at[b]], rows_v.at[b], gsem.at[b])

    def body(j, carry):
      slot = lax.rem(j, NBUF)
      pltpu.make_async_copy(
          table_hbm.at[idx_v.at[j]], rows_v.at[slot], gsem.at[slot]).wait()
      pltpu.sync_copy(
          rows_v.at[slot],
          out_hbm.at[pl.ds(base + j * CHUNK, CHUNK), pl.ds(0, dim)])
      nxt = j + NBUF
      pltpu.async_copy(table_hbm.at[idx_v.at[nxt]], rows_v.at[slot],
                       gsem.at[slot])
      return carry

    lax.fori_loop(0, n_chunks - NBUF, body, 0)

    for b in range(NBUF):
      j = n_chunks - NBUF + b
      slot = j % NBUF
      pltpu.make_async_copy(
          table_hbm.at[idx_v.at[j]], rows_v.at[slot], gsem.at[slot]).wait()
      pltpu.sync_copy(
          rows_v.at[slot],
          out_hbm.at[pl.ds(base + j * CHUNK, CHUNK), pl.ds(0, dim)])

  return gather_kernel


def kernel(input_ids, attention_mask, emb_weight):
  batch, seq = input_ids.shape
  vocab, dim = emb_weight.shape
  n_ids = batch * seq
  table_wide = jax.lax.optimization_barrier(emb_weight.reshape(vocab // 2, 2 * dim))
  table = table_wide.reshape(vocab, dim)
  ids = input_ids.reshape(NUM_WORKERS, n_ids // (NUM_WORKERS * CHUNK), CHUNK)
  ids = ids.astype(jnp.int32)
  out128 = _make_gather(n_ids, vocab, 128)(ids, table)
  return out128[:, :dim].reshape(batch, seq, dim), attention_mask
